# Initial kernel scaffold; baseline (speedup 1.0000x reference)
import jax
import jax.numpy as jnp
from jax.experimental import pallas as pl


def kernel(x, emb_table):
    raise NotImplementedError("write your pallas kernel here")


# speedup vs baseline: 81.0684x; 81.0684x over previous
import jax
import jax.numpy as jnp
from jax.experimental import pallas as pl


def _body(x_ref, emb_ref, o_ref):
    n_rows = emb_ref.shape[0]
    mean = jnp.sum(emb_ref[...], axis=0, keepdims=True) * (1.0 / n_rows)
    o_ref[...] = x_ref[...] + mean


def kernel(x, emb_table):
    B, S, D = x.shape
    R = B * S
    xf = x.reshape(R, D)
    BLK = 1024
    out = pl.pallas_call(
        _body,
        grid=(R // BLK,),
        in_specs=[
            pl.BlockSpec((BLK, D), lambda i: (i, 0)),
            pl.BlockSpec(emb_table.shape, lambda i: (0, 0)),
        ],
        out_specs=pl.BlockSpec((BLK, D), lambda i: (i, 0)),
        out_shape=jax.ShapeDtypeStruct((R, D), x.dtype),
    )(xf, emb_table)
    return out.reshape(B, S, D)


# speedup vs baseline: 87.4207x; 1.0784x over previous
import jax
import jax.numpy as jnp
from jax.experimental import pallas as pl


def _body(x_ref, emb_ref, o_ref):
    n_rows = emb_ref.shape[0]
    mean = jnp.sum(emb_ref[...], axis=0, keepdims=True) * (1.0 / n_rows)
    o_ref[...] = x_ref[...] + mean


def kernel(x, emb_table):
    B, S, D = x.shape
    R = B * S
    xf = x.reshape(R, D)
    BLK = 2048
    out = pl.pallas_call(
        _body,
        grid=(R // BLK,),
        in_specs=[
            pl.BlockSpec((BLK, D), lambda i: (i, 0)),
            pl.BlockSpec(emb_table.shape, lambda i: (0, 0)),
        ],
        out_specs=pl.BlockSpec((BLK, D), lambda i: (i, 0)),
        out_shape=jax.ShapeDtypeStruct((R, D), x.dtype),
    )(xf, emb_table)
    return out.reshape(B, S, D)


# speedup vs baseline: 93.2335x; 1.0665x over previous
import jax
import jax.numpy as jnp
from jax.experimental import pallas as pl


def _body(x_ref, emb_ref, o_ref):
    n_rows = emb_ref.shape[0]
    mean = jnp.sum(emb_ref[...], axis=0, keepdims=True) * (1.0 / n_rows)
    o_ref[...] = x_ref[...] + mean


def kernel(x, emb_table):
    B, S, D = x.shape
    R = B * S
    xf = x.reshape(R, D)
    BLK = 4096
    out = pl.pallas_call(
        _body,
        grid=(R // BLK,),
        in_specs=[
            pl.BlockSpec((BLK, D), lambda i: (i, 0)),
            pl.BlockSpec(emb_table.shape, lambda i: (0, 0)),
        ],
        out_specs=pl.BlockSpec((BLK, D), lambda i: (i, 0)),
        out_shape=jax.ShapeDtypeStruct((R, D), x.dtype),
    )(xf, emb_table)
    return out.reshape(B, S, D)
